# baseline (speedup 1.0000x reference)
import functools
import jax
import jax.numpy as jnp
from jax import lax
from jax.experimental import pallas as pl
from jax.experimental.pallas import tpu as pltpu
from jax.experimental.pallas import tpu_sc as plsc

NV = 10000
NE = 10000
PAD = 163840
ROWS = 10240
TRASH = 10000
CHUNK = 128
NTILES = 16
CPT = PAD // (NTILES * CHUNK)
RPT = ROWS // NTILES
IB = 10

_MESH = plsc.VectorSubcoreMesh(core_axis_name="c", subcore_axis_name="s")
NW = 32
CPW = PAD // (NW * CHUNK)


def _sc_pass(D, table_rows):
    @functools.partial(
        pl.kernel,
        mesh=_MESH,
        out_type=jax.ShapeDtypeStruct((2, ROWS, D), jnp.float32),
        scratch_types=[
            pltpu.VMEM((2, IB, 2, CHUNK), jnp.int32),
            pltpu.VMEM((2, CHUNK, D), jnp.float32),
            pltpu.VMEM_SHARED((ROWS, D), jnp.float32),
            pltpu.SemaphoreType.DMA((2,)),
            pltpu.SemaphoreType.DMA((2,)),
            pltpu.SemaphoreType.DMA((2,)),
        ],
    )
    def k(tstack, idx3, zeros, out, ib, buf, acc, isem, gsem, ssem):
        cid = lax.axis_index("c")
        sid = lax.axis_index("s")

        pltpu.sync_copy(zeros.at[pl.ds(sid * RPT, RPT)],
                        acc.at[pl.ds(sid * RPT, RPT)])
        plsc.subcore_barrier()

        _pipe(tstack.at[cid], acc, idx3, ib, buf, isem, gsem, ssem,
              sid * CPT, CPT // IB)
        plsc.subcore_barrier()
        pltpu.sync_copy(acc.at[pl.ds(sid * RPT, RPT)],
                        out.at[cid, pl.ds(sid * RPT, RPT)])

    return k


def _pipe(tbl, acc, idx3, ib, buf, isem, gsem, ssem, off, nblk):
    def refill(s, kb):
        pltpu.async_copy(idx3.at[pl.ds(off + kb * IB, IB)], ib.at[s],
                         isem.at[s])

    def iwait(s):
        pltpu.make_async_copy(idx3.at[pl.ds(off, IB)], ib.at[s],
                              isem.at[s]).wait()

    def gwait(b):
        pltpu.make_async_copy(tbl.at[ib.at[0, 0, 0]], buf.at[b],
                              gsem.at[b]).wait()

    def swait(b):
        pltpu.make_async_copy(buf.at[b], acc.at[ib.at[0, 0, 1]],
                              ssem.at[b]).wait()

    def block(s, kb):
        iwait(s)
        for b in range(2):
            pltpu.async_copy(tbl.at[ib.at[s, b, 0]], buf.at[b], gsem.at[b])
        for grp in range(IB // 2):
            for b in range(2):
                c = grp * 2 + b
                gwait(b)
                pltpu.async_copy(buf.at[b], acc.at[ib.at[s, c, 1]],
                                 ssem.at[b], add=True)
            for b in range(2):
                swait(b)
                nc = grp * 2 + b + 2
                if nc < IB:
                    pltpu.async_copy(tbl.at[ib.at[s, nc, 0]], buf.at[b],
                                     gsem.at[b])
        refill(s, jnp.minimum(kb + 2, nblk - 1))

    refill(0, 0)
    refill(1, min(1, nblk - 1))

    def outer(g2, carry):
        block(0, 2 * g2)
        block(1, 2 * g2 + 1)
        return carry

    lax.fori_loop(0, nblk // 2, outer, 0)
    iwait(0)
    iwait(1)


@functools.partial(
    pl.kernel,
    mesh=_MESH,
    out_type=jax.ShapeDtypeStruct((2, ROWS, 128), jnp.float32),
    scratch_types=[
        pltpu.VMEM((2, IB, 2, CHUNK), jnp.int32),
        pltpu.VMEM((2, CHUNK, 128), jnp.float32),
        pltpu.VMEM_SHARED((ROWS, 128), jnp.float32),
        pltpu.SemaphoreType.DMA((2,)),
        pltpu.SemaphoreType.DMA((2,)),
        pltpu.SemaphoreType.DMA((2,)),
    ],
)
def _sc_pass_pairs(tbl, idx3, zeros, out, ib, buf, acc, isem, gsem, ssem):
    cid = lax.axis_index("c")
    sid = lax.axis_index("s")
    wid = sid * 2 + cid

    pltpu.sync_copy(zeros.at[pl.ds(sid * RPT, RPT)],
                    acc.at[pl.ds(sid * RPT, RPT)])
    plsc.subcore_barrier()

    _pipe(tbl, acc, idx3, ib, buf, isem, gsem, ssem, wid * CPW, CPW // IB)
    plsc.subcore_barrier()
    pltpu.sync_copy(acc.at[pl.ds(sid * RPT, RPT)],
                    out.at[cid, pl.ds(sid * RPT, RPT)])


@functools.partial(
    pl.kernel,
    mesh=_MESH,
    out_type=[
        jax.ShapeDtypeStruct((ROWS, 128), jnp.float32),
        jax.ShapeDtypeStruct((ROWS, 128), jnp.float32),
    ],
    scratch_types=[
        pltpu.VMEM((CPT, CHUNK), jnp.int32),
        pltpu.VMEM((CHUNK, 128), jnp.float32),
        pltpu.VMEM_SHARED((ROWS, 128), jnp.float32),
        pltpu.SemaphoreType.DMA,
    ],
)
def _sc_counts(e_sidx, v_sidx, ones, zeros, out_e, out_v, sv, buf, acc, sem):
    cid = lax.axis_index("c")
    sid = lax.axis_index("s")

    pltpu.sync_copy(ones, buf)
    pltpu.sync_copy(zeros.at[pl.ds(sid * RPT, RPT)],
                    acc.at[pl.ds(sid * RPT, RPT)])

    def run(sidx, out):
        pltpu.sync_copy(sidx.at[pl.ds(sid * CPT, CPT)], sv)
        plsc.subcore_barrier()

        def step(it, carry):
            base = it * 8
            for b in range(8):
                pltpu.async_copy(buf, acc.at[sv.at[base + b]], sem, add=True)
            for b in range(8):
                pltpu.make_async_copy(buf, acc.at[sv.at[0]], sem).wait()
            return carry

        lax.fori_loop(0, CPT // 8, step, 0)
        plsc.subcore_barrier()
        pltpu.sync_copy(acc.at[pl.ds(sid * RPT, RPT)],
                        out.at[pl.ds(sid * RPT, RPT)])

    @pl.when(cid == 0)
    def _():
        run(e_sidx, out_e)

    @pl.when(cid == 1)
    def _():
        run(v_sidx, out_v)


_BM = 256
_GRID = (ROWS // _BM,)


def _row_block(D):
    return pl.BlockSpec((_BM, D), lambda i: (i, 0))


def _full_block(shape):
    return pl.BlockSpec(shape, lambda i: (0,) * len(shape))


def _scale_body(e, cnt, f):
    inv = 1.0 / jnp.maximum(cnt[:, 0:1], 1.0)
    f[0] = e[0] * inv
    f[1] = e[1] * inv


def _tc_scale(D):
    return pl.pallas_call(
        _scale_body,
        grid=_GRID,
        in_specs=[pl.BlockSpec((2, _BM, D), lambda i: (0, i, 0)),
                  _row_block(128)],
        out_specs=pl.BlockSpec((2, _BM, D), lambda i: (0, i, 0)),
        out_shape=jax.ShapeDtypeStruct((2, ROWS, D), jnp.float32),
    )


def _pair_block():
    return pl.BlockSpec((2, _BM, 128), lambda i: (0, i, 0))


def _scale2_body(p, cnt, f):
    inv = 1.0 / jnp.maximum(cnt[:, 0:1], 1.0)
    f[...] = (p[0] + p[1]) * inv


_tc_scale2 = pl.pallas_call(
    _scale2_body,
    grid=_GRID,
    in_specs=[_pair_block(), _row_block(128)],
    out_specs=_row_block(128),
    out_shape=jax.ShapeDtypeStruct((ROWS, 128), jnp.float32),
)


def _mm_body(vs, cnt, w1a, w1b, b1, w2, b2, y):
    c = cnt[:, 0:1]
    inv = 1.0 / jnp.maximum(c, 1.0)
    maskb = jnp.where(c > 0.0, 1.0, 0.0) * b1[0:1, :]
    t = jnp.dot(vs[0] * inv, w1a[...], preferred_element_type=jnp.float32)
    t += jnp.dot(vs[1] * inv, w1b[...], preferred_element_type=jnp.float32)
    h = jnp.maximum(t + maskb, 0.0)
    y2 = jnp.dot(h, w2[...], preferred_element_type=jnp.float32) + b2[0:1, :]
    y[...] = jnp.concatenate([y2, jnp.zeros((_BM, 64), jnp.float32)], axis=1)


_tc_matmul = pl.pallas_call(
    _mm_body,
    grid=_GRID,
    in_specs=[
        pl.BlockSpec((2, _BM, 128), lambda i: (0, i, 0)), _row_block(128),
        _full_block((128, 512)), _full_block((128, 512)),
        _full_block((8, 512)), _full_block((512, 64)), _full_block((8, 64)),
    ],
    out_specs=_row_block(128),
    out_shape=jax.ShapeDtypeStruct((ROWS, 128), jnp.float32),
)


def _final_body(q, cnt, out):
    inv = 1.0 / jnp.maximum(cnt[:, 0:1], 1.0)
    out[...] = ((q[0] + q[1]) * inv)[:, :64]


_tc_final = pl.pallas_call(
    _final_body,
    grid=_GRID,
    in_specs=[_pair_block(), _row_block(128)],
    out_specs=_row_block(64),
    out_shape=jax.ShapeDtypeStruct((ROWS, 64), jnp.float32),
)


def kernel(X, hyperedge_index, W1, b1, W2, b2):
    v_idx = hyperedge_index[0].astype(jnp.int32)
    e_idx = hyperedge_index[1].astype(jnp.int32)
    nnz = v_idx.shape[0]
    npad = PAD - nnz
    zpad = jnp.zeros((npad,), jnp.int32)
    tpad = TRASH + (jnp.arange(npad, dtype=jnp.int32) % (ROWS - TRASH))
    v_g = jnp.concatenate([v_idx, zpad]).reshape(NTILES * CPT, CHUNK)
    v_s = jnp.concatenate([v_idx, tpad]).reshape(NTILES * CPT, CHUNK)
    e_g = jnp.concatenate([e_idx, zpad]).reshape(NTILES * CPT, CHUNK)
    e_s = jnp.concatenate([e_idx, tpad]).reshape(NTILES * CPT, CHUNK)

    z128 = jnp.zeros((ROWS, 128), jnp.float32)
    ones128 = jnp.zeros((CHUNK, 128), jnp.float32).at[:, 0].set(1.0)

    e_cnt, v_cnt = _sc_counts(e_s, v_s, ones128, z128)
    z1 = jnp.minimum(e_cnt, 0.0)

    Xs = X.reshape(NV, 2, 128).transpose(1, 0, 2)
    ve3 = jnp.stack([v_g, e_s], axis=1)
    ev3 = jnp.stack([e_g, v_s], axis=1)
    es = _sc_pass(128, NV)(Xs, ve3, z1)
    ef = _tc_scale(128)(es, e_cnt)
    vs = _sc_pass(128, ROWS)(ef, ev3, z128)

    b1r = jnp.broadcast_to(b1[None, :], (8, 512))
    b2r = jnp.broadcast_to(b2[None, :], (8, 64))
    y = _tc_matmul(vs, v_cnt, W1[:128], W1[128:], b1r, W2, b2r)

    ep = _sc_pass_pairs(y, ve3, z128)
    ef2 = _tc_scale2(ep, e_cnt)
    vp = _sc_pass_pairs(ef2, ev3, z128)
    out = _tc_final(vp, v_cnt)
    return out[:NV]


# speedup vs baseline: 1.1354x; 1.1354x over previous
import functools
import jax
import jax.numpy as jnp
from jax import lax
from jax.experimental import pallas as pl
from jax.experimental.pallas import tpu as pltpu
from jax.experimental.pallas import tpu_sc as plsc

NV = 10000
NE = 10000
PAD = 163840
ROWS = 10240
TRASH = 10000
CHUNK = 128
NTILES = 16
CPT = PAD // (NTILES * CHUNK)
RPT = ROWS // NTILES
IB = 10

_MESH = plsc.VectorSubcoreMesh(core_axis_name="c", subcore_axis_name="s")
NW = 32
CPW = PAD // (NW * CHUNK)


def _sc_pass(D, table_rows):
    @functools.partial(
        pl.kernel,
        mesh=_MESH,
        out_type=jax.ShapeDtypeStruct((2, ROWS, D), jnp.float32),
        scratch_types=[
            pltpu.VMEM((2, IB, 2, CHUNK), jnp.int32),
            pltpu.VMEM((2, CHUNK, D), jnp.float32),
            pltpu.VMEM_SHARED((ROWS, D), jnp.float32),
            pltpu.SemaphoreType.DMA((2,)),
            pltpu.SemaphoreType.DMA((2,)),
            pltpu.SemaphoreType.DMA((2,)),
        ],
    )
    def k(tstack, idx3, zeros, out, ib, buf, acc, isem, gsem, ssem):
        cid = lax.axis_index("c")
        sid = lax.axis_index("s")

        pltpu.sync_copy(zeros.at[pl.ds(sid * RPT, RPT)],
                        acc.at[pl.ds(sid * RPT, RPT)])
        plsc.subcore_barrier()

        _pipe(tstack.at[cid], acc, idx3, ib, buf, isem, gsem, ssem,
              sid * CPT, CPT // IB)
        plsc.subcore_barrier()
        pltpu.sync_copy(acc.at[pl.ds(sid * RPT, RPT)],
                        out.at[cid, pl.ds(sid * RPT, RPT)])

    return k


def _pipe(tbl, acc, idx3, ib, buf, isem, gsem, ssem, off, nblk):
    def refill(s, kb):
        pltpu.async_copy(idx3.at[pl.ds(off + kb * IB, IB)], ib.at[s],
                         isem.at[s])

    def iwait(s):
        pltpu.make_async_copy(idx3.at[pl.ds(off, IB)], ib.at[s],
                              isem.at[s]).wait()

    def gwait(b):
        pltpu.make_async_copy(tbl.at[ib.at[0, 0, 0]], buf.at[b],
                              gsem.at[b]).wait()

    def swait(b):
        pltpu.make_async_copy(buf.at[b], acc.at[ib.at[0, 0, 1]],
                              ssem.at[b]).wait()

    def block(s, kb):
        iwait(s)
        for b in range(2):
            pltpu.async_copy(tbl.at[ib.at[s, b, 0]], buf.at[b], gsem.at[b])
        for grp in range(IB // 2):
            for b in range(2):
                c = grp * 2 + b
                gwait(b)
                pltpu.async_copy(buf.at[b], acc.at[ib.at[s, c, 1]],
                                 ssem.at[b], add=True)
            for b in range(2):
                swait(b)
                nc = grp * 2 + b + 2
                if nc < IB:
                    pltpu.async_copy(tbl.at[ib.at[s, nc, 0]], buf.at[b],
                                     gsem.at[b])
        refill(s, jnp.minimum(kb + 2, nblk - 1))

    refill(0, 0)
    refill(1, jnp.minimum(1, nblk - 1))

    def outer(g2, carry):
        block(0, 2 * g2)
        block(1, 2 * g2 + 1)
        return carry

    lax.fori_loop(0, nblk // 2, outer, 0)
    iwait(0)
    iwait(1)


@functools.partial(
    pl.kernel,
    mesh=_MESH,
    out_type=jax.ShapeDtypeStruct((2, ROWS, 128), jnp.float32),
    scratch_types=[
        pltpu.VMEM((2, IB, 2, CHUNK), jnp.int32),
        pltpu.VMEM((2, CHUNK, 128), jnp.float32),
        pltpu.VMEM_SHARED((ROWS, 128), jnp.float32),
        pltpu.SemaphoreType.DMA((2,)),
        pltpu.SemaphoreType.DMA((2,)),
        pltpu.SemaphoreType.DMA((2,)),
    ],
)
def _sc_pass_pairs(tbl, idx3, zeros, out, ib, buf, acc, isem, gsem, ssem):
    cid = lax.axis_index("c")
    sid = lax.axis_index("s")
    wid = sid * 2 + cid

    pltpu.sync_copy(zeros.at[pl.ds(sid * RPT, RPT)],
                    acc.at[pl.ds(sid * RPT, RPT)])
    plsc.subcore_barrier()

    nblk = jnp.where(cid == 0, 6, 2)
    off = jnp.where(cid == 0, sid * (6 * IB), 16 * 6 * IB + sid * (2 * IB))
    _pipe(tbl, acc, idx3, ib, buf, isem, gsem, ssem, off, nblk)
    plsc.subcore_barrier()
    pltpu.sync_copy(acc.at[pl.ds(sid * RPT, RPT)],
                    out.at[cid, pl.ds(sid * RPT, RPT)])


@functools.partial(
    pl.kernel,
    mesh=_MESH,
    out_type=[
        jax.ShapeDtypeStruct((ROWS, 128), jnp.float32),
        jax.ShapeDtypeStruct((ROWS, 128), jnp.float32),
    ],
    scratch_types=[
        pltpu.VMEM((CPT, CHUNK), jnp.int32),
        pltpu.VMEM((CHUNK, 128), jnp.float32),
        pltpu.VMEM_SHARED((ROWS, 128), jnp.float32),
        pltpu.SemaphoreType.DMA,
    ],
)
def _sc_counts(e_sidx, v_sidx, ones, zeros, out_e, out_v, sv, buf, acc, sem):
    cid = lax.axis_index("c")
    sid = lax.axis_index("s")

    pltpu.sync_copy(ones, buf)
    pltpu.sync_copy(zeros.at[pl.ds(sid * RPT, RPT)],
                    acc.at[pl.ds(sid * RPT, RPT)])

    def run(sidx, out):
        pltpu.sync_copy(sidx.at[pl.ds(sid * CPT, CPT)], sv)
        plsc.subcore_barrier()

        def step(it, carry):
            base = it * 8
            for b in range(8):
                pltpu.async_copy(buf, acc.at[sv.at[base + b]], sem, add=True)
            for b in range(8):
                pltpu.make_async_copy(buf, acc.at[sv.at[0]], sem).wait()
            return carry

        lax.fori_loop(0, CPT // 8, step, 0)
        plsc.subcore_barrier()
        pltpu.sync_copy(acc.at[pl.ds(sid * RPT, RPT)],
                        out.at[pl.ds(sid * RPT, RPT)])

    @pl.when(cid == 0)
    def _():
        run(e_sidx, out_e)

    @pl.when(cid == 1)
    def _():
        run(v_sidx, out_v)


_BM = 256
_GRID = (ROWS // _BM,)


def _row_block(D):
    return pl.BlockSpec((_BM, D), lambda i: (i, 0))


def _full_block(shape):
    return pl.BlockSpec(shape, lambda i: (0,) * len(shape))


def _scale_body(e, cnt, f):
    inv = 1.0 / jnp.maximum(cnt[:, 0:1], 1.0)
    f[0] = e[0] * inv
    f[1] = e[1] * inv


def _tc_scale(D):
    return pl.pallas_call(
        _scale_body,
        grid=_GRID,
        in_specs=[pl.BlockSpec((2, _BM, D), lambda i: (0, i, 0)),
                  _row_block(128)],
        out_specs=pl.BlockSpec((2, _BM, D), lambda i: (0, i, 0)),
        out_shape=jax.ShapeDtypeStruct((2, ROWS, D), jnp.float32),
    )


def _pair_block():
    return pl.BlockSpec((2, _BM, 128), lambda i: (0, i, 0))


def _scale2_body(p, cnt, f):
    inv = 1.0 / jnp.maximum(cnt[:, 0:1], 1.0)
    f[...] = (p[0] + p[1]) * inv


_tc_scale2 = pl.pallas_call(
    _scale2_body,
    grid=_GRID,
    in_specs=[_pair_block(), _row_block(128)],
    out_specs=_row_block(128),
    out_shape=jax.ShapeDtypeStruct((ROWS, 128), jnp.float32),
)


def _mm_body(vs, cnt, w1a, w1b, b1, w2, b2, y):
    c = cnt[:, 0:1]
    inv = 1.0 / jnp.maximum(c, 1.0)
    maskb = jnp.where(c > 0.0, 1.0, 0.0) * b1[0:1, :]
    t = jnp.dot(vs[0] * inv, w1a[...], preferred_element_type=jnp.float32)
    t += jnp.dot(vs[1] * inv, w1b[...], preferred_element_type=jnp.float32)
    h = jnp.maximum(t + maskb, 0.0)
    y2 = jnp.dot(h, w2[...], preferred_element_type=jnp.float32) + b2[0:1, :]
    y[...] = jnp.concatenate([y2, jnp.zeros((_BM, 64), jnp.float32)], axis=1)


_tc_matmul = pl.pallas_call(
    _mm_body,
    grid=_GRID,
    in_specs=[
        pl.BlockSpec((2, _BM, 128), lambda i: (0, i, 0)), _row_block(128),
        _full_block((128, 512)), _full_block((128, 512)),
        _full_block((8, 512)), _full_block((512, 64)), _full_block((8, 64)),
    ],
    out_specs=_row_block(128),
    out_shape=jax.ShapeDtypeStruct((ROWS, 128), jnp.float32),
)


def _final_body(q, cnt, out):
    inv = 1.0 / jnp.maximum(cnt[:, 0:1], 1.0)
    out[...] = ((q[0] + q[1]) * inv)[:, :64]


_tc_final = pl.pallas_call(
    _final_body,
    grid=_GRID,
    in_specs=[_pair_block(), _row_block(128)],
    out_specs=_row_block(64),
    out_shape=jax.ShapeDtypeStruct((ROWS, 64), jnp.float32),
)


def kernel(X, hyperedge_index, W1, b1, W2, b2):
    v_idx = hyperedge_index[0].astype(jnp.int32)
    e_idx = hyperedge_index[1].astype(jnp.int32)
    nnz = v_idx.shape[0]
    npad = PAD - nnz
    zpad = jnp.zeros((npad,), jnp.int32)
    tpad = jnp.full((npad,), TRASH, jnp.int32)
    v_g = jnp.concatenate([v_idx, zpad]).reshape(NTILES * CPT, CHUNK)
    v_s = jnp.concatenate([v_idx, tpad]).reshape(NTILES * CPT, CHUNK)
    e_g = jnp.concatenate([e_idx, zpad]).reshape(NTILES * CPT, CHUNK)
    e_s = jnp.concatenate([e_idx, tpad]).reshape(NTILES * CPT, CHUNK)

    z128 = jnp.zeros((ROWS, 128), jnp.float32)
    ones128 = jnp.zeros((CHUNK, 128), jnp.float32).at[:, 0].set(1.0)

    e_cnt, v_cnt = _sc_counts(e_s, v_s, ones128, z128)
    z1 = jnp.minimum(e_cnt, 0.0)

    Xs = X.reshape(NV, 2, 128).transpose(1, 0, 2)
    ve3 = jnp.stack([v_g, e_s], axis=1)
    ev3 = jnp.stack([e_g, v_s], axis=1)
    es = _sc_pass(128, NV)(Xs, ve3, z1)
    ef = _tc_scale(128)(es, e_cnt)
    vs = _sc_pass(128, ROWS)(ef, ev3, z128)

    b1r = jnp.broadcast_to(b1[None, :], (8, 512))
    b2r = jnp.broadcast_to(b2[None, :], (8, 64))
    y = _tc_matmul(vs, v_cnt, W1[:128], W1[128:], b1r, W2, b2r)

    ep = _sc_pass_pairs(y, ve3, z128)
    ef2 = _tc_scale2(ep, e_cnt)
    vp = _sc_pass_pairs(ef2, ev3, z128)
    out = _tc_final(vp, v_cnt)
    return out[:NV]
